# baseline (speedup 1.0000x reference)
import functools

import jax
import jax.numpy as jnp
from jax import lax
from jax.experimental import pallas as pl
from jax.experimental.pallas import tpu as pltpu
from jax.experimental.pallas import tpu_sc as plsc

N = 10000
E = 320000
HD = 128
HH = 64
ZD = 64

NC = 2
NS = 16
CHUNK = 640
EPAD = 327680
NCHUNKS = EPAD // CHUNK
TRASH = 10200
NPAD = 10240
ROWS_PER_TILE = NPAD // NS

F32 = jnp.float32


def _f32(shape):
    return jax.ShapeDtypeStruct(shape, F32)


@functools.lru_cache(maxsize=None)
def _build_sc_kernels():
  mesh = plsc.VectorSubcoreMesh(
      core_axis_name="c", subcore_axis_name="s",
      num_cores=NC, num_subcores=NS)
  cparams = pltpu.CompilerParams(
      use_tc_tiling_on_sc=False, needs_layout_passes=False)

  @functools.partial(
      pl.kernel,
      out_type=(_f32((NPAD, HH)), _f32((NPAD, HH))),
      mesh=mesh,
      compiler_params=cparams,
      scratch_types=(
          pltpu.VMEM_SHARED((NPAD, HH), F32),
          pltpu.VMEM((2, CHUNK), jnp.int32),
          pltpu.VMEM((CHUNK, HH), F32),
          pltpu.VMEM((ROWS_PER_TILE, HH), F32),
          pltpu.SemaphoreType.DMA,
      ),
  )
  def sc_agg(feat2, ei_hbm, z64_hbm, out_lo, out_hi,
             acc, ei_v, rows, dbuf, sem):
    c = lax.axis_index("c")
    s = lax.axis_index("s")
    rstart = s * ROWS_PER_TILE
    pltpu.sync_copy(z64_hbm, dbuf)
    pltpu.sync_copy(dbuf, acc.at[pl.ds(rstart, ROWS_PER_TILE)])
    plsc.subcore_barrier()

    cpt = NCHUNKS // NS
    coff = c * N

    def body(i, _):
        pltpu.sync_copy(ei_hbm.at[s * cpt + i], ei_v)
        for j in range(CHUNK // 16):
            sl = pl.ds(j * 16, 16)
            ei_v[0, sl] = ei_v[0, sl] + coff
        pltpu.async_copy(feat2.at[ei_v.at[0]], rows, sem).wait()
        pltpu.sync_copy(rows, acc.at[ei_v.at[1]], add=True)
        return 0

    lax.fori_loop(0, cpt, body, 0)
    plsc.subcore_barrier()

    pltpu.sync_copy(acc.at[pl.ds(rstart, ROWS_PER_TILE)], dbuf)

    @pl.when(c == 0)
    def _():
        pltpu.sync_copy(dbuf, out_lo.at[pl.ds(rstart, ROWS_PER_TILE)])

    @pl.when(c == 1)
    def _():
        pltpu.sync_copy(dbuf, out_hi.at[pl.ds(rstart, ROWS_PER_TILE)])

  @functools.partial(
      pl.kernel,
      out_type=_f32((NPAD, 16)),
      mesh=mesh,
      compiler_params=cparams,
      scratch_types=(
          pltpu.VMEM_SHARED((NPAD, 16), F32),
          pltpu.VMEM((2, CHUNK), jnp.int32),
          pltpu.VMEM((CHUNK, 16), F32),
          pltpu.VMEM((ROWS_PER_TILE, 16), F32),
      ),
  )
  def sc_deg(ei_hbm, z16_hbm, ones_hbm, out_deg,
             dacc, ei_v, ones_v, dbuf16):
    c = lax.axis_index("c")
    s = lax.axis_index("s")
    rstart = s * ROWS_PER_TILE
    pltpu.sync_copy(z16_hbm, dbuf16)
    pltpu.sync_copy(dbuf16, dacc.at[pl.ds(rstart, ROWS_PER_TILE)])
    pltpu.sync_copy(ones_hbm, ones_v)
    plsc.subcore_barrier()

    cpt = NCHUNKS // NS

    def body(i, _):
        pltpu.sync_copy(ei_hbm.at[s * cpt + i], ei_v)
        pltpu.sync_copy(ones_v, dacc.at[ei_v.at[1]], add=True)
        return 0

    lax.fori_loop(0, cpt, body, 0)
    plsc.subcore_barrier()

    @pl.when(c == 0)
    def _():
        pltpu.sync_copy(dacc.at[pl.ds(rstart, ROWS_PER_TILE)], dbuf16)
        pltpu.sync_copy(dbuf16, out_deg.at[pl.ds(rstart, ROWS_PER_TILE)])

  return sc_agg, sc_deg


def _sc_seg(feat_lo, feat_hi, ei_r):
    feat2 = jnp.concatenate([feat_lo, feat_hi], axis=0)
    z64 = jnp.zeros((ROWS_PER_TILE, HH), F32)
    lo, hi = _build_sc_kernels()[0](feat2, ei_r, z64)
    return lo[:N], hi[:N]


def _sc_deg(ei_r):
    z16 = jnp.zeros((ROWS_PER_TILE, 16), F32)
    ones16 = jnp.ones((CHUNK, 16), F32)
    deg = _build_sc_kernels()[1](ei_r, z16, ones16)
    return deg[:N, :1]


RB = 1000


def _row_spec(w):
    return pl.BlockSpec((RB, w), lambda i: (i, 0))


def _full_spec(shape):
    nd = len(shape)
    return pl.BlockSpec(shape, lambda i: (0,) * nd)


def _rowcall(body, n_out, out_w, ins):
    specs = []
    for a, rowwise in ins:
        specs.append(_row_spec(a.shape[1]) if rowwise else _full_spec(a.shape))
    outs = tuple(_f32((N, w)) for w in out_w)
    out_specs = tuple(_row_spec(w) for w in out_w)
    if n_out == 1:
        outs, out_specs = outs[0], out_specs[0]
    return pl.pallas_call(
        body, grid=(N // RB,), in_specs=specs,
        out_specs=out_specs, out_shape=outs,
    )(*[a for a, _ in ins])


def _cc(lo_ref, hi_ref):
    return jnp.concatenate([lo_ref[...], hi_ref[...]], axis=1)


def _tc_phiX(x, W, b):
    def body(x_r, W_r, b_r, lo_r, hi_r):
        y = jax.nn.relu(jnp.dot(x_r[...], W_r[...]) + b_r[...])
        lo_r[...] = y[:, :HH]
        hi_r[...] = y[:, HH:]
    return _rowcall(body, 2, (HH, HH), [(x, True), (W, False), (b, False)])


def _tc_encx(Ap_lo, Ap_hi, Ah_lo, Ah_hi, deg, px_lo, px_hi, hl_lo, hl_hi,
             Wl, bl, Wr):
    def body(Apl, Aph, Ahl_, Ahh, deg_r, pxl, pxh, hll, hlh,
             Wl_r, bl_r, Wr_r, lo_r, hi_r):
        d = jnp.maximum(deg_r[...], 1.0)
        m = jnp.concatenate([_cc(Apl, Aph) / d, _cc(Ahl_, Ahh) / d], axis=1)
        xx = jnp.concatenate([_cc(pxl, pxh), _cc(hll, hlh)], axis=1)
        y = jax.nn.relu(jnp.dot(m, Wl_r[...]) + bl_r[...]
                        + jnp.dot(xx, Wr_r[...]))
        lo_r[...] = y[:, :HH]
        hi_r[...] = y[:, HH:]
    return _rowcall(body, 2, (HH, HH), [
        (Ap_lo, True), (Ap_hi, True), (Ah_lo, True), (Ah_hi, True),
        (deg, True), (px_lo, True), (px_hi, True), (hl_lo, True),
        (hl_hi, True), (Wl, False), (bl, False), (Wr, False)])


def _tc_latent(Ae_lo, Ae_hi, deg, ex_lo, ex_hi, hl_lo, hl_hi, eps,
               em_Wl, em_b, em_Wr, es_Wl, es_b, es_Wr,
               pr_W, pr_b, pm_W, pm_b, ps_W, ps_b, pz_W, pz_b):
    def body(Ael, Aeh, deg_r, exl, exh, hll, hlh, eps_r,
             emWl, emb, emWr, esWl, esb, esWr, prW, prb, pmW, pmb,
             psW, psb, pzW, pzb,
             em_o, es_o, pm_o, ps_o, z_o, pz_lo, pz_hi):
        d = jnp.maximum(deg_r[...], 1.0)
        m = _cc(Ael, Aeh) / d
        ex = _cc(exl, exh)
        hl = _cc(hll, hlh)
        enc_mean = jnp.dot(m, emWl[...]) + emb[...] + jnp.dot(ex, emWr[...])
        enc_std = jax.nn.softplus(
            jnp.dot(m, esWl[...]) + esb[...] + jnp.dot(ex, esWr[...]))
        px = jax.nn.relu(jnp.dot(hl, prW[...]) + prb[...])
        pm_o[...] = jnp.dot(px, pmW[...]) + pmb[...]
        ps_o[...] = jax.nn.softplus(jnp.dot(px, psW[...]) + psb[...])
        z = eps_r[...] * enc_std + enc_mean
        em_o[...] = enc_mean
        es_o[...] = enc_std
        z_o[...] = z
        phiZ = jax.nn.relu(jnp.dot(z, pzW[...]) + pzb[...])
        pz_lo[...] = phiZ[:, :HH]
        pz_hi[...] = phiZ[:, HH:]
    return _rowcall(body, 7, (ZD, ZD, ZD, ZD, ZD, HH, HH), [
        (Ae_lo, True), (Ae_hi, True), (deg, True), (ex_lo, True),
        (ex_hi, True), (hl_lo, True), (hl_hi, True), (eps, True),
        (em_Wl, False), (em_b, False), (em_Wr, False),
        (es_Wl, False), (es_b, False), (es_Wr, False), (pr_W, False),
        (pr_b, False), (pm_W, False), (pm_b, False), (ps_W, False),
        (ps_b, False), (pz_W, False), (pz_b, False)])


def _tc_gates(Ap_lo, Ap_hi, Az_lo, Az_hi, Ah_lo, Ah_hi, deg,
              px_lo, px_hi, pz_lo, pz_hi, hl_lo, hl_hi, wz, wr):
    def body(Apl, Aph, Azl, Azh, Ahl_, Ahh, deg_r,
             pxl, pxh, pzl, pzh, hll, hlh,
             zWl, zb, zWr, zhWl, zhb, zhWr,
             rWl, rb, rWr, rhWl, rhb, rhWr,
             zg_o, rh_lo, rh_hi):
        d = jnp.maximum(deg_r[...], 1.0)
        m_rnn = jnp.concatenate([_cc(Apl, Aph) / d, _cc(Azl, Azh) / d],
                                axis=1)
        mh = _cc(Ahl_, Ahh) / d
        rnn_in = jnp.concatenate([_cc(pxl, pxh), _cc(pzl, pzh)], axis=1)
        hl = _cc(hll, hlh)

        def gate(Wl, b, Wr, hWl, hb, hWr):
            return jax.nn.sigmoid(
                jnp.dot(m_rnn, Wl[...]) + b[...] + jnp.dot(rnn_in, Wr[...])
                + jnp.dot(mh, hWl[...]) + hb[...] + jnp.dot(hl, hWr[...]))

        z_g = gate(zWl, zb, zWr, zhWl, zhb, zhWr)
        r_g = gate(rWl, rb, rWr, rhWl, rhb, rhWr)
        zg_o[...] = z_g
        rh = r_g * hl
        rh_lo[...] = rh[:, :HH]
        rh_hi[...] = rh[:, HH:]
    return _rowcall(body, 3, (HD, HH, HH), [
        (Ap_lo, True), (Ap_hi, True), (Az_lo, True), (Az_hi, True),
        (Ah_lo, True), (Ah_hi, True), (deg, True),
        (px_lo, True), (px_hi, True), (pz_lo, True), (pz_hi, True),
        (hl_lo, True), (hl_hi, True),
        *[(w, False) for w in wz], *[(w, False) for w in wr]])


def _tc_hout(Ap_lo, Ap_hi, Az_lo, Az_hi, Ar_lo, Ar_hi, deg,
             px_lo, px_hi, pz_lo, pz_hi, rh_lo, rh_hi, zg, hl_lo, hl_hi,
             wh):
    def body(Apl, Aph, Azl, Azh, Arl, Arh, deg_r,
             pxl, pxh, pzl, pzh, rhl, rhh, zg_r, hll, hlh,
             hWl, hb, hWr, hhWl, hhb, hhWr, o_r):
        d = jnp.maximum(deg_r[...], 1.0)
        m_rnn = jnp.concatenate([_cc(Apl, Aph) / d, _cc(Azl, Azh) / d],
                                axis=1)
        mrh = _cc(Arl, Arh) / d
        rnn_in = jnp.concatenate([_cc(pxl, pxh), _cc(pzl, pzh)], axis=1)
        rh = _cc(rhl, rhh)
        hl = _cc(hll, hlh)
        h_t = jnp.tanh(
            jnp.dot(m_rnn, hWl[...]) + hb[...] + jnp.dot(rnn_in, hWr[...])
            + jnp.dot(mrh, hhWl[...]) + hhb[...] + jnp.dot(rh, hhWr[...]))
        z_g = zg_r[...]
        o_r[...] = z_g * hl + (1.0 - z_g) * h_t
    return _rowcall(body, 1, (HD,), [
        (Ap_lo, True), (Ap_hi, True), (Az_lo, True), (Az_hi, True),
        (Ar_lo, True), (Ar_hi, True), (deg, True),
        (px_lo, True), (px_hi, True), (pz_lo, True), (pz_hi, True),
        (rh_lo, True), (rh_hi, True), (zg, True), (hl_lo, True),
        (hl_hi, True), *[(w, False) for w in wh]])


ADJ_BI = 1024
ADJ_BJ = 2048


def _tc_adj(z):
    def body(zi_r, zj_r, o_r):
        o_r[...] = jax.nn.sigmoid(
            lax.dot_general(zi_r[...], zj_r[...], (((1,), (1,)), ((), ()))))
    return pl.pallas_call(
        body, grid=(pl.cdiv(N, ADJ_BI), pl.cdiv(N, ADJ_BJ)),
        in_specs=[pl.BlockSpec((ADJ_BI, ZD), lambda i, j: (i, 0)),
                  pl.BlockSpec((ADJ_BJ, ZD), lambda i, j: (j, 0))],
        out_specs=pl.BlockSpec((ADJ_BI, ADJ_BJ), lambda i, j: (i, j)),
        out_shape=_f32((N, N)),
    )(z, z)


def kernel(x, h, edge_index, eps_noise, params):
    p = params
    hl_lo = h[0, :, :HH]
    hl_hi = h[0, :, HH:]
    pad = EPAD - E
    src_p = jnp.concatenate([edge_index[0], jnp.zeros((pad,), jnp.int32)])
    dst_p = jnp.concatenate(
        [edge_index[1], jnp.full((pad,), TRASH, jnp.int32)])
    ei_r = jnp.stack([src_p.reshape(NCHUNKS, CHUNK),
                      dst_p.reshape(NCHUNKS, CHUNK)], axis=1)

    def b2(v):
        return v.reshape(1, -1)

    px_lo, px_hi = _tc_phiX(x, p["phi_x_W"], b2(p["phi_x_b"]))

    deg = _sc_deg(ei_r)
    Ap_lo, Ap_hi = _sc_seg(px_lo, px_hi, ei_r)
    Ah_lo, Ah_hi = _sc_seg(hl_lo, hl_hi, ei_r)

    ex_lo, ex_hi = _tc_encx(Ap_lo, Ap_hi, Ah_lo, Ah_hi, deg,
                            px_lo, px_hi, hl_lo, hl_hi,
                            p["enc_Wl"], b2(p["enc_bl"]), p["enc_Wr"])

    Ae_lo, Ae_hi = _sc_seg(ex_lo, ex_hi, ei_r)

    enc_mean, enc_std, prior_mean, prior_std, z, pz_lo, pz_hi = _tc_latent(
        Ae_lo, Ae_hi, deg, ex_lo, ex_hi, hl_lo, hl_hi, eps_noise,
        p["em_Wl"], b2(p["em_bl"]), p["em_Wr"],
        p["es_Wl"], b2(p["es_bl"]), p["es_Wr"],
        p["prior_W"], b2(p["prior_b"]),
        p["pm_W"], b2(p["pm_b"]), p["ps_W"], b2(p["ps_b"]),
        p["phi_z_W"], b2(p["phi_z_b"]))

    Az_lo, Az_hi = _sc_seg(pz_lo, pz_hi, ei_r)

    wz = [p["xz_Wl"], b2(p["xz_bl"]), p["xz_Wr"],
          p["hz_Wl"], b2(p["hz_bl"]), p["hz_Wr"]]
    wr = [p["xr_Wl"], b2(p["xr_bl"]), p["xr_Wr"],
          p["hr_Wl"], b2(p["hr_bl"]), p["hr_Wr"]]
    z_g, rh_lo, rh_hi = _tc_gates(Ap_lo, Ap_hi, Az_lo, Az_hi, Ah_lo, Ah_hi,
                                  deg, px_lo, px_hi, pz_lo, pz_hi,
                                  hl_lo, hl_hi, wz, wr)

    Ar_lo, Ar_hi = _sc_seg(rh_lo, rh_hi, ei_r)

    wh = [p["xh_Wl"], b2(p["xh_bl"]), p["xh_Wr"],
          p["hh_Wl"], b2(p["hh_bl"]), p["hh_Wr"]]
    out = _tc_hout(Ap_lo, Ap_hi, Az_lo, Az_hi, Ar_lo, Ar_hi, deg,
                   px_lo, px_hi, pz_lo, pz_hi, rh_lo, rh_hi, z_g,
                   hl_lo, hl_hi, wh)

    adj = _tc_adj(z)

    return (adj, prior_mean, prior_std, enc_mean, enc_std, z, out[None])


# speedup vs baseline: 1.6284x; 1.6284x over previous
import functools

import jax
import jax.numpy as jnp
from jax import lax
from jax.experimental import pallas as pl
from jax.experimental.pallas import tpu as pltpu
from jax.experimental.pallas import tpu_sc as plsc

N = 10000
E = 320000
HD = 128
HH = 64
ZD = 64

NC = 2
NS = 16
CHUNK = 400
DCH = 400
NPAD = 10240
ROWS_PER_TILE = NPAD // NS

F32 = jnp.float32


def _f32(shape):
    return jax.ShapeDtypeStruct(shape, F32)


@functools.lru_cache(maxsize=None)
def _build_sc_kernels():
  mesh = plsc.VectorSubcoreMesh(
      core_axis_name="c", subcore_axis_name="s",
      num_cores=NC, num_subcores=NS)
  cparams = pltpu.CompilerParams(
      use_tc_tiling_on_sc=False, needs_layout_passes=False)

  @functools.partial(
      pl.kernel,
      out_type=(_f32((NPAD, HH)), _f32((NPAD, HH))),
      mesh=mesh,
      compiler_params=cparams,
      scratch_types=(
          pltpu.VMEM_SHARED((NPAD, HH), F32),
          pltpu.VMEM((CHUNK,), jnp.int32),
          pltpu.VMEM((CHUNK,), jnp.int32),
          pltpu.VMEM((CHUNK, HH), F32),
          pltpu.VMEM((ROWS_PER_TILE, HH), F32),
          pltpu.SemaphoreType.DMA,
      ),
  )
  def sc_agg(feat2, src_hbm, dst_hbm, z64_hbm, out_lo, out_hi,
             acc, src_v, dst_v, rows, dbuf, sem):
    c = lax.axis_index("c")
    s = lax.axis_index("s")
    rstart = s * ROWS_PER_TILE
    pltpu.sync_copy(z64_hbm, dbuf)
    pltpu.sync_copy(dbuf, acc.at[pl.ds(rstart, ROWS_PER_TILE)])
    plsc.subcore_barrier()

    per_tile = E // NS
    base = s * per_tile
    coff = c * N

    def body(i, _):
        off = pl.multiple_of(base + i * CHUNK, 8)
        pltpu.sync_copy(src_hbm.at[pl.ds(off, CHUNK)], src_v)
        pltpu.sync_copy(dst_hbm.at[pl.ds(off, CHUNK)], dst_v)
        for j in range(CHUNK // 16):
            sl = pl.ds(j * 16, 16)
            src_v[sl] = src_v[sl] + coff
        pltpu.async_copy(feat2.at[src_v], rows, sem).wait()
        pltpu.sync_copy(rows, acc.at[dst_v], add=True)
        return 0

    lax.fori_loop(0, per_tile // CHUNK, body, 0)
    plsc.subcore_barrier()

    pltpu.sync_copy(acc.at[pl.ds(rstart, ROWS_PER_TILE)], dbuf)

    @pl.when(c == 0)
    def _():
        pltpu.sync_copy(dbuf, out_lo.at[pl.ds(rstart, ROWS_PER_TILE)])

    @pl.when(c == 1)
    def _():
        pltpu.sync_copy(dbuf, out_hi.at[pl.ds(rstart, ROWS_PER_TILE)])

  @functools.partial(
      pl.kernel,
      out_type=_f32((NC, NPAD)),
      mesh=mesh,
      compiler_params=cparams,
      scratch_types=(
          pltpu.VMEM_SHARED((NS, NPAD), F32),
          pltpu.VMEM((NPAD,), F32),
          pltpu.VMEM((DCH,), jnp.int32),
          pltpu.VMEM((ROWS_PER_TILE,), F32),
          pltpu.VMEM((ROWS_PER_TILE,), F32),
      ),
  )
  def sc_deg(dst_hbm, out_deg, slots, hist, dst_v, sumbuf, tmp):
    c = lax.axis_index("c")
    s = lax.axis_index("s")
    zero16 = jnp.zeros((16,), F32)
    one16 = jnp.ones((16,), F32)

    def zbody(i, _):
        hist[pl.ds(i * 16, 16)] = zero16
        return 0

    lax.fori_loop(0, NPAD // 16, zbody, 0)

    per_tile = E // (NC * NS)
    base = (c * NS + s) * per_tile

    def body(i, _):
        off = pl.multiple_of(base + i * DCH, 8)
        pltpu.sync_copy(dst_hbm.at[pl.ds(off, DCH)], dst_v)
        for k in range(DCH // 16):
            idx = dst_v[pl.ds(k * 16, 16)]
            plsc.addupdate_scatter(hist, [idx], one16)
        return 0

    lax.fori_loop(0, per_tile // DCH, body, 0)
    pltpu.sync_copy(hist, slots.at[s])
    plsc.subcore_barrier()

    rstart = s * ROWS_PER_TILE
    pltpu.sync_copy(slots.at[0, pl.ds(rstart, ROWS_PER_TILE)], sumbuf)
    for t in range(1, NS):
        pltpu.sync_copy(slots.at[t, pl.ds(rstart, ROWS_PER_TILE)], tmp)
        for j in range(ROWS_PER_TILE // 16):
            sl = pl.ds(j * 16, 16)
            sumbuf[sl] = sumbuf[sl] + tmp[sl]
    pltpu.sync_copy(sumbuf, out_deg.at[c, pl.ds(rstart, ROWS_PER_TILE)])

  return sc_agg, sc_deg


def _sc_seg(feat_lo, feat_hi, src_e, dst_e):
    feat2 = jnp.concatenate([feat_lo, feat_hi], axis=0)
    z64 = jnp.zeros((ROWS_PER_TILE, HH), F32)
    lo, hi = _build_sc_kernels()[0](feat2, src_e, dst_e, z64)
    return lo[:N], hi[:N]


def _sc_deg(dst_e):
    degp = _build_sc_kernels()[1](dst_e)
    return (degp[0] + degp[1])[:N, None]


RB = 1000


def _row_spec(w):
    return pl.BlockSpec((RB, w), lambda i: (i, 0))


def _full_spec(shape):
    nd = len(shape)
    return pl.BlockSpec(shape, lambda i: (0,) * nd)


def _rowcall(body, n_out, out_w, ins):
    specs = []
    for a, rowwise in ins:
        specs.append(_row_spec(a.shape[1]) if rowwise else _full_spec(a.shape))
    outs = tuple(_f32((N, w)) for w in out_w)
    out_specs = tuple(_row_spec(w) for w in out_w)
    if n_out == 1:
        outs, out_specs = outs[0], out_specs[0]
    return pl.pallas_call(
        body, grid=(N // RB,), in_specs=specs,
        out_specs=out_specs, out_shape=outs,
    )(*[a for a, _ in ins])


def _cc(lo_ref, hi_ref):
    return jnp.concatenate([lo_ref[...], hi_ref[...]], axis=1)


def _tc_phiX(x, W, b):
    def body(x_r, W_r, b_r, lo_r, hi_r):
        y = jax.nn.relu(jnp.dot(x_r[...], W_r[...]) + b_r[...])
        lo_r[...] = y[:, :HH]
        hi_r[...] = y[:, HH:]
    return _rowcall(body, 2, (HH, HH), [(x, True), (W, False), (b, False)])


def _tc_encx(Ap_lo, Ap_hi, Ah_lo, Ah_hi, deg, px_lo, px_hi, hl_lo, hl_hi,
             Wl, bl, Wr):
    def body(Apl, Aph, Ahl_, Ahh, deg_r, pxl, pxh, hll, hlh,
             Wl_r, bl_r, Wr_r, lo_r, hi_r):
        d = jnp.maximum(deg_r[...], 1.0)
        m = jnp.concatenate([_cc(Apl, Aph) / d, _cc(Ahl_, Ahh) / d], axis=1)
        xx = jnp.concatenate([_cc(pxl, pxh), _cc(hll, hlh)], axis=1)
        y = jax.nn.relu(jnp.dot(m, Wl_r[...]) + bl_r[...]
                        + jnp.dot(xx, Wr_r[...]))
        lo_r[...] = y[:, :HH]
        hi_r[...] = y[:, HH:]
    return _rowcall(body, 2, (HH, HH), [
        (Ap_lo, True), (Ap_hi, True), (Ah_lo, True), (Ah_hi, True),
        (deg, True), (px_lo, True), (px_hi, True), (hl_lo, True),
        (hl_hi, True), (Wl, False), (bl, False), (Wr, False)])


def _tc_latent(Ae_lo, Ae_hi, deg, ex_lo, ex_hi, hl_lo, hl_hi, eps,
               em_Wl, em_b, em_Wr, es_Wl, es_b, es_Wr,
               pr_W, pr_b, pm_W, pm_b, ps_W, ps_b, pz_W, pz_b):
    def body(Ael, Aeh, deg_r, exl, exh, hll, hlh, eps_r,
             emWl, emb, emWr, esWl, esb, esWr, prW, prb, pmW, pmb,
             psW, psb, pzW, pzb,
             em_o, es_o, pm_o, ps_o, z_o, pz_lo, pz_hi):
        d = jnp.maximum(deg_r[...], 1.0)
        m = _cc(Ael, Aeh) / d
        ex = _cc(exl, exh)
        hl = _cc(hll, hlh)
        enc_mean = jnp.dot(m, emWl[...]) + emb[...] + jnp.dot(ex, emWr[...])
        enc_std = jax.nn.softplus(
            jnp.dot(m, esWl[...]) + esb[...] + jnp.dot(ex, esWr[...]))
        px = jax.nn.relu(jnp.dot(hl, prW[...]) + prb[...])
        pm_o[...] = jnp.dot(px, pmW[...]) + pmb[...]
        ps_o[...] = jax.nn.softplus(jnp.dot(px, psW[...]) + psb[...])
        z = eps_r[...] * enc_std + enc_mean
        em_o[...] = enc_mean
        es_o[...] = enc_std
        z_o[...] = z
        phiZ = jax.nn.relu(jnp.dot(z, pzW[...]) + pzb[...])
        pz_lo[...] = phiZ[:, :HH]
        pz_hi[...] = phiZ[:, HH:]
    return _rowcall(body, 7, (ZD, ZD, ZD, ZD, ZD, HH, HH), [
        (Ae_lo, True), (Ae_hi, True), (deg, True), (ex_lo, True),
        (ex_hi, True), (hl_lo, True), (hl_hi, True), (eps, True),
        (em_Wl, False), (em_b, False), (em_Wr, False),
        (es_Wl, False), (es_b, False), (es_Wr, False), (pr_W, False),
        (pr_b, False), (pm_W, False), (pm_b, False), (ps_W, False),
        (ps_b, False), (pz_W, False), (pz_b, False)])


def _tc_gates(Ap_lo, Ap_hi, Az_lo, Az_hi, Ah_lo, Ah_hi, deg,
              px_lo, px_hi, pz_lo, pz_hi, hl_lo, hl_hi, wz, wr):
    def body(Apl, Aph, Azl, Azh, Ahl_, Ahh, deg_r,
             pxl, pxh, pzl, pzh, hll, hlh,
             zWl, zb, zWr, zhWl, zhb, zhWr,
             rWl, rb, rWr, rhWl, rhb, rhWr,
             zg_o, rh_lo, rh_hi):
        d = jnp.maximum(deg_r[...], 1.0)
        m_rnn = jnp.concatenate([_cc(Apl, Aph) / d, _cc(Azl, Azh) / d],
                                axis=1)
        mh = _cc(Ahl_, Ahh) / d
        rnn_in = jnp.concatenate([_cc(pxl, pxh), _cc(pzl, pzh)], axis=1)
        hl = _cc(hll, hlh)

        def gate(Wl, b, Wr, hWl, hb, hWr):
            return jax.nn.sigmoid(
                jnp.dot(m_rnn, Wl[...]) + b[...] + jnp.dot(rnn_in, Wr[...])
                + jnp.dot(mh, hWl[...]) + hb[...] + jnp.dot(hl, hWr[...]))

        z_g = gate(zWl, zb, zWr, zhWl, zhb, zhWr)
        r_g = gate(rWl, rb, rWr, rhWl, rhb, rhWr)
        zg_o[...] = z_g
        rh = r_g * hl
        rh_lo[...] = rh[:, :HH]
        rh_hi[...] = rh[:, HH:]
    return _rowcall(body, 3, (HD, HH, HH), [
        (Ap_lo, True), (Ap_hi, True), (Az_lo, True), (Az_hi, True),
        (Ah_lo, True), (Ah_hi, True), (deg, True),
        (px_lo, True), (px_hi, True), (pz_lo, True), (pz_hi, True),
        (hl_lo, True), (hl_hi, True),
        *[(w, False) for w in wz], *[(w, False) for w in wr]])


def _tc_hout(Ap_lo, Ap_hi, Az_lo, Az_hi, Ar_lo, Ar_hi, deg,
             px_lo, px_hi, pz_lo, pz_hi, rh_lo, rh_hi, zg, hl_lo, hl_hi,
             wh):
    def body(Apl, Aph, Azl, Azh, Arl, Arh, deg_r,
             pxl, pxh, pzl, pzh, rhl, rhh, zg_r, hll, hlh,
             hWl, hb, hWr, hhWl, hhb, hhWr, o_r):
        d = jnp.maximum(deg_r[...], 1.0)
        m_rnn = jnp.concatenate([_cc(Apl, Aph) / d, _cc(Azl, Azh) / d],
                                axis=1)
        mrh = _cc(Arl, Arh) / d
        rnn_in = jnp.concatenate([_cc(pxl, pxh), _cc(pzl, pzh)], axis=1)
        rh = _cc(rhl, rhh)
        hl = _cc(hll, hlh)
        h_t = jnp.tanh(
            jnp.dot(m_rnn, hWl[...]) + hb[...] + jnp.dot(rnn_in, hWr[...])
            + jnp.dot(mrh, hhWl[...]) + hhb[...] + jnp.dot(rh, hhWr[...]))
        z_g = zg_r[...]
        o_r[...] = z_g * hl + (1.0 - z_g) * h_t
    return _rowcall(body, 1, (HD,), [
        (Ap_lo, True), (Ap_hi, True), (Az_lo, True), (Az_hi, True),
        (Ar_lo, True), (Ar_hi, True), (deg, True),
        (px_lo, True), (px_hi, True), (pz_lo, True), (pz_hi, True),
        (rh_lo, True), (rh_hi, True), (zg, True), (hl_lo, True),
        (hl_hi, True), *[(w, False) for w in wh]])


ADJ_BI = 1024
ADJ_BJ = 2048


def _tc_adj(z):
    def body(zi_r, zj_r, o_r):
        o_r[...] = jax.nn.sigmoid(
            lax.dot_general(zi_r[...], zj_r[...], (((1,), (1,)), ((), ()))))
    return pl.pallas_call(
        body, grid=(pl.cdiv(N, ADJ_BI), pl.cdiv(N, ADJ_BJ)),
        in_specs=[pl.BlockSpec((ADJ_BI, ZD), lambda i, j: (i, 0)),
                  pl.BlockSpec((ADJ_BJ, ZD), lambda i, j: (j, 0))],
        out_specs=pl.BlockSpec((ADJ_BI, ADJ_BJ), lambda i, j: (i, j)),
        out_shape=_f32((N, N)),
    )(z, z)


def kernel(x, h, edge_index, eps_noise, params):
    p = params
    hl_lo = h[0, :, :HH]
    hl_hi = h[0, :, HH:]
    src_e = edge_index[0]
    dst_e = edge_index[1]

    def b2(v):
        return v.reshape(1, -1)

    px_lo, px_hi = _tc_phiX(x, p["phi_x_W"], b2(p["phi_x_b"]))

    deg = _sc_deg(dst_e)
    Ap_lo, Ap_hi = _sc_seg(px_lo, px_hi, src_e, dst_e)
    Ah_lo, Ah_hi = _sc_seg(hl_lo, hl_hi, src_e, dst_e)

    ex_lo, ex_hi = _tc_encx(Ap_lo, Ap_hi, Ah_lo, Ah_hi, deg,
                            px_lo, px_hi, hl_lo, hl_hi,
                            p["enc_Wl"], b2(p["enc_bl"]), p["enc_Wr"])

    Ae_lo, Ae_hi = _sc_seg(ex_lo, ex_hi, src_e, dst_e)

    enc_mean, enc_std, prior_mean, prior_std, z, pz_lo, pz_hi = _tc_latent(
        Ae_lo, Ae_hi, deg, ex_lo, ex_hi, hl_lo, hl_hi, eps_noise,
        p["em_Wl"], b2(p["em_bl"]), p["em_Wr"],
        p["es_Wl"], b2(p["es_bl"]), p["es_Wr"],
        p["prior_W"], b2(p["prior_b"]),
        p["pm_W"], b2(p["pm_b"]), p["ps_W"], b2(p["ps_b"]),
        p["phi_z_W"], b2(p["phi_z_b"]))

    Az_lo, Az_hi = _sc_seg(pz_lo, pz_hi, src_e, dst_e)

    wz = [p["xz_Wl"], b2(p["xz_bl"]), p["xz_Wr"],
          p["hz_Wl"], b2(p["hz_bl"]), p["hz_Wr"]]
    wr = [p["xr_Wl"], b2(p["xr_bl"]), p["xr_Wr"],
          p["hr_Wl"], b2(p["hr_bl"]), p["hr_Wr"]]
    z_g, rh_lo, rh_hi = _tc_gates(Ap_lo, Ap_hi, Az_lo, Az_hi, Ah_lo, Ah_hi,
                                  deg, px_lo, px_hi, pz_lo, pz_hi,
                                  hl_lo, hl_hi, wz, wr)

    Ar_lo, Ar_hi = _sc_seg(rh_lo, rh_hi, src_e, dst_e)

    wh = [p["xh_Wl"], b2(p["xh_bl"]), p["xh_Wr"],
          p["hh_Wl"], b2(p["hh_bl"]), p["hh_Wr"]]
    out = _tc_hout(Ap_lo, Ap_hi, Az_lo, Az_hi, Ar_lo, Ar_hi, deg,
                   px_lo, px_hi, pz_lo, pz_hi, rh_lo, rh_hi, z_g,
                   hl_lo, hl_hi, wh)

    adj = _tc_adj(z)

    return (adj, prior_mean, prior_std, enc_mean, enc_std, z, out[None])


# speedup vs baseline: 1.6689x; 1.0248x over previous
import functools

import jax
import jax.numpy as jnp
from jax import lax
from jax.experimental import pallas as pl
from jax.experimental.pallas import tpu as pltpu
from jax.experimental.pallas import tpu_sc as plsc

N = 10000
E = 320000
HD = 128
HH = 64
ZD = 64

NC = 2
NS = 16
CHUNK = 400
DCH = 400
NPAD = 10240
ROWS_PER_TILE = NPAD // NS

F32 = jnp.float32


def _f32(shape):
    return jax.ShapeDtypeStruct(shape, F32)


@functools.lru_cache(maxsize=None)
def _build_sc_kernels():
  mesh = plsc.VectorSubcoreMesh(
      core_axis_name="c", subcore_axis_name="s",
      num_cores=NC, num_subcores=NS)
  cparams = pltpu.CompilerParams(
      use_tc_tiling_on_sc=False, needs_layout_passes=False)

  @functools.partial(
      pl.kernel,
      out_type=(_f32((NPAD, HH)), _f32((NPAD, HH))),
      mesh=mesh,
      compiler_params=cparams,
      scratch_types=(
          pltpu.VMEM_SHARED((NPAD, HH), F32),
          pltpu.VMEM((CHUNK,), jnp.int32),
          pltpu.VMEM((CHUNK,), jnp.int32),
          pltpu.VMEM((CHUNK, HH), F32),
          pltpu.VMEM((ROWS_PER_TILE, HH), F32),
          pltpu.SemaphoreType.DMA,
      ),
  )
  def sc_agg(feat2, src_hbm, dst_hbm, out_lo, out_hi,
             acc, src_v, dst_v, rows, dbuf, sem):
    c = lax.axis_index("c")
    s = lax.axis_index("s")
    rstart = s * ROWS_PER_TILE
    zero16 = jnp.zeros((16,), F32)

    def zbody(r, _):
        for q in range(HH // 16):
            dbuf[r, pl.ds(q * 16, 16)] = zero16
        return 0

    lax.fori_loop(0, ROWS_PER_TILE, zbody, 0)
    pltpu.sync_copy(dbuf, acc.at[pl.ds(rstart, ROWS_PER_TILE)])
    plsc.subcore_barrier()

    per_tile = E // NS
    base = s * per_tile
    coff = c * N

    def body(i, _):
        off = pl.multiple_of(base + i * CHUNK, 8)
        pltpu.sync_copy(src_hbm.at[pl.ds(off, CHUNK)], src_v)
        pltpu.sync_copy(dst_hbm.at[pl.ds(off, CHUNK)], dst_v)
        for j in range(CHUNK // 16):
            sl = pl.ds(j * 16, 16)
            src_v[sl] = src_v[sl] + coff
        pltpu.async_copy(feat2.at[src_v], rows, sem).wait()
        pltpu.sync_copy(rows, acc.at[dst_v], add=True)
        return 0

    lax.fori_loop(0, per_tile // CHUNK, body, 0)
    plsc.subcore_barrier()

    pltpu.sync_copy(acc.at[pl.ds(rstart, ROWS_PER_TILE)], dbuf)

    @pl.when(c == 0)
    def _():
        pltpu.sync_copy(dbuf, out_lo.at[pl.ds(rstart, ROWS_PER_TILE)])

    @pl.when(c == 1)
    def _():
        pltpu.sync_copy(dbuf, out_hi.at[pl.ds(rstart, ROWS_PER_TILE)])

  @functools.partial(
      pl.kernel,
      out_type=_f32((NC, NPAD)),
      mesh=mesh,
      compiler_params=cparams,
      scratch_types=(
          pltpu.VMEM_SHARED((NS, NPAD), F32),
          pltpu.VMEM((NPAD,), F32),
          pltpu.VMEM((DCH,), jnp.int32),
          pltpu.VMEM((ROWS_PER_TILE,), F32),
          pltpu.VMEM((ROWS_PER_TILE,), F32),
      ),
  )
  def sc_deg(dst_hbm, out_deg, slots, hist, dst_v, sumbuf, tmp):
    c = lax.axis_index("c")
    s = lax.axis_index("s")
    zero16 = jnp.zeros((16,), F32)
    one16 = jnp.ones((16,), F32)

    def zbody(i, _):
        hist[pl.ds(i * 16, 16)] = zero16
        return 0

    lax.fori_loop(0, NPAD // 16, zbody, 0)

    per_tile = E // (NC * NS)
    base = (c * NS + s) * per_tile

    def body(i, _):
        off = pl.multiple_of(base + i * DCH, 8)
        pltpu.sync_copy(dst_hbm.at[pl.ds(off, DCH)], dst_v)
        for k in range(DCH // 16):
            idx = dst_v[pl.ds(k * 16, 16)]
            plsc.addupdate_scatter(hist, [idx], one16)
        return 0

    lax.fori_loop(0, per_tile // DCH, body, 0)
    pltpu.sync_copy(hist, slots.at[s])
    plsc.subcore_barrier()

    rstart = s * ROWS_PER_TILE
    pltpu.sync_copy(slots.at[0, pl.ds(rstart, ROWS_PER_TILE)], sumbuf)
    for t in range(1, NS):
        pltpu.sync_copy(slots.at[t, pl.ds(rstart, ROWS_PER_TILE)], tmp)
        for j in range(ROWS_PER_TILE // 16):
            sl = pl.ds(j * 16, 16)
            sumbuf[sl] = sumbuf[sl] + tmp[sl]
    pltpu.sync_copy(sumbuf, out_deg.at[c, pl.ds(rstart, ROWS_PER_TILE)])

  return sc_agg, sc_deg


def _sc_seg(feat_lo, feat_hi, src_e, dst_e):
    feat2 = jnp.concatenate([feat_lo, feat_hi], axis=0)
    lo, hi = _build_sc_kernels()[0](feat2, src_e, dst_e)
    return lo[:N], hi[:N]


def _sc_deg(dst_e):
    degp = _build_sc_kernels()[1](dst_e)
    return (degp[0] + degp[1])[:N, None]


RB = 1000


def _row_spec(w):
    return pl.BlockSpec((RB, w), lambda i: (i, 0))


def _full_spec(shape):
    nd = len(shape)
    return pl.BlockSpec(shape, lambda i: (0,) * nd)


def _rowcall(body, n_out, out_w, ins):
    specs = []
    for a, rowwise in ins:
        specs.append(_row_spec(a.shape[1]) if rowwise else _full_spec(a.shape))
    outs = tuple(_f32((N, w)) for w in out_w)
    out_specs = tuple(_row_spec(w) for w in out_w)
    if n_out == 1:
        outs, out_specs = outs[0], out_specs[0]
    return pl.pallas_call(
        body, grid=(N // RB,), in_specs=specs,
        out_specs=out_specs, out_shape=outs,
    )(*[a for a, _ in ins])


def _cc(lo_ref, hi_ref):
    return jnp.concatenate([lo_ref[...], hi_ref[...]], axis=1)


def _tc_phiX(x, W, b):
    def body(x_r, W_r, b_r, lo_r, hi_r):
        y = jax.nn.relu(jnp.dot(x_r[...], W_r[...]) + b_r[...])
        lo_r[...] = y[:, :HH]
        hi_r[...] = y[:, HH:]
    return _rowcall(body, 2, (HH, HH), [(x, True), (W, False), (b, False)])


def _tc_encx(Ap_lo, Ap_hi, Ah_lo, Ah_hi, deg, px_lo, px_hi, hl_lo, hl_hi,
             Wl, bl, Wr):
    def body(Apl, Aph, Ahl_, Ahh, deg_r, pxl, pxh, hll, hlh,
             Wl_r, bl_r, Wr_r, lo_r, hi_r):
        d = jnp.maximum(deg_r[...], 1.0)
        m = jnp.concatenate([_cc(Apl, Aph) / d, _cc(Ahl_, Ahh) / d], axis=1)
        xx = jnp.concatenate([_cc(pxl, pxh), _cc(hll, hlh)], axis=1)
        y = jax.nn.relu(jnp.dot(m, Wl_r[...]) + bl_r[...]
                        + jnp.dot(xx, Wr_r[...]))
        lo_r[...] = y[:, :HH]
        hi_r[...] = y[:, HH:]
    return _rowcall(body, 2, (HH, HH), [
        (Ap_lo, True), (Ap_hi, True), (Ah_lo, True), (Ah_hi, True),
        (deg, True), (px_lo, True), (px_hi, True), (hl_lo, True),
        (hl_hi, True), (Wl, False), (bl, False), (Wr, False)])


def _tc_latent(Ae_lo, Ae_hi, deg, ex_lo, ex_hi, hl_lo, hl_hi, eps,
               em_Wl, em_b, em_Wr, es_Wl, es_b, es_Wr,
               pr_W, pr_b, pm_W, pm_b, ps_W, ps_b, pz_W, pz_b):
    def body(Ael, Aeh, deg_r, exl, exh, hll, hlh, eps_r,
             emWl, emb, emWr, esWl, esb, esWr, prW, prb, pmW, pmb,
             psW, psb, pzW, pzb,
             em_o, es_o, pm_o, ps_o, z_o, pz_lo, pz_hi):
        d = jnp.maximum(deg_r[...], 1.0)
        m = _cc(Ael, Aeh) / d
        ex = _cc(exl, exh)
        hl = _cc(hll, hlh)
        enc_mean = jnp.dot(m, emWl[...]) + emb[...] + jnp.dot(ex, emWr[...])
        enc_std = jax.nn.softplus(
            jnp.dot(m, esWl[...]) + esb[...] + jnp.dot(ex, esWr[...]))
        px = jax.nn.relu(jnp.dot(hl, prW[...]) + prb[...])
        pm_o[...] = jnp.dot(px, pmW[...]) + pmb[...]
        ps_o[...] = jax.nn.softplus(jnp.dot(px, psW[...]) + psb[...])
        z = eps_r[...] * enc_std + enc_mean
        em_o[...] = enc_mean
        es_o[...] = enc_std
        z_o[...] = z
        phiZ = jax.nn.relu(jnp.dot(z, pzW[...]) + pzb[...])
        pz_lo[...] = phiZ[:, :HH]
        pz_hi[...] = phiZ[:, HH:]
    return _rowcall(body, 7, (ZD, ZD, ZD, ZD, ZD, HH, HH), [
        (Ae_lo, True), (Ae_hi, True), (deg, True), (ex_lo, True),
        (ex_hi, True), (hl_lo, True), (hl_hi, True), (eps, True),
        (em_Wl, False), (em_b, False), (em_Wr, False),
        (es_Wl, False), (es_b, False), (es_Wr, False), (pr_W, False),
        (pr_b, False), (pm_W, False), (pm_b, False), (ps_W, False),
        (ps_b, False), (pz_W, False), (pz_b, False)])


def _tc_gates(Ap_lo, Ap_hi, Az_lo, Az_hi, Ah_lo, Ah_hi, deg,
              px_lo, px_hi, pz_lo, pz_hi, hl_lo, hl_hi, wz, wr):
    def body(Apl, Aph, Azl, Azh, Ahl_, Ahh, deg_r,
             pxl, pxh, pzl, pzh, hll, hlh,
             zWl, zb, zWr, zhWl, zhb, zhWr,
             rWl, rb, rWr, rhWl, rhb, rhWr,
             zg_o, rh_lo, rh_hi):
        d = jnp.maximum(deg_r[...], 1.0)
        m_rnn = jnp.concatenate([_cc(Apl, Aph) / d, _cc(Azl, Azh) / d],
                                axis=1)
        mh = _cc(Ahl_, Ahh) / d
        rnn_in = jnp.concatenate([_cc(pxl, pxh), _cc(pzl, pzh)], axis=1)
        hl = _cc(hll, hlh)

        def gate(Wl, b, Wr, hWl, hb, hWr):
            return jax.nn.sigmoid(
                jnp.dot(m_rnn, Wl[...]) + b[...] + jnp.dot(rnn_in, Wr[...])
                + jnp.dot(mh, hWl[...]) + hb[...] + jnp.dot(hl, hWr[...]))

        z_g = gate(zWl, zb, zWr, zhWl, zhb, zhWr)
        r_g = gate(rWl, rb, rWr, rhWl, rhb, rhWr)
        zg_o[...] = z_g
        rh = r_g * hl
        rh_lo[...] = rh[:, :HH]
        rh_hi[...] = rh[:, HH:]
    return _rowcall(body, 3, (HD, HH, HH), [
        (Ap_lo, True), (Ap_hi, True), (Az_lo, True), (Az_hi, True),
        (Ah_lo, True), (Ah_hi, True), (deg, True),
        (px_lo, True), (px_hi, True), (pz_lo, True), (pz_hi, True),
        (hl_lo, True), (hl_hi, True),
        *[(w, False) for w in wz], *[(w, False) for w in wr]])


def _tc_hout(Ap_lo, Ap_hi, Az_lo, Az_hi, Ar_lo, Ar_hi, deg,
             px_lo, px_hi, pz_lo, pz_hi, rh_lo, rh_hi, zg, hl_lo, hl_hi,
             wh):
    def body(Apl, Aph, Azl, Azh, Arl, Arh, deg_r,
             pxl, pxh, pzl, pzh, rhl, rhh, zg_r, hll, hlh,
             hWl, hb, hWr, hhWl, hhb, hhWr, o_r):
        d = jnp.maximum(deg_r[...], 1.0)
        m_rnn = jnp.concatenate([_cc(Apl, Aph) / d, _cc(Azl, Azh) / d],
                                axis=1)
        mrh = _cc(Arl, Arh) / d
        rnn_in = jnp.concatenate([_cc(pxl, pxh), _cc(pzl, pzh)], axis=1)
        rh = _cc(rhl, rhh)
        hl = _cc(hll, hlh)
        h_t = jnp.tanh(
            jnp.dot(m_rnn, hWl[...]) + hb[...] + jnp.dot(rnn_in, hWr[...])
            + jnp.dot(mrh, hhWl[...]) + hhb[...] + jnp.dot(rh, hhWr[...]))
        z_g = zg_r[...]
        o_r[...] = z_g * hl + (1.0 - z_g) * h_t
    return _rowcall(body, 1, (HD,), [
        (Ap_lo, True), (Ap_hi, True), (Az_lo, True), (Az_hi, True),
        (Ar_lo, True), (Ar_hi, True), (deg, True),
        (px_lo, True), (px_hi, True), (pz_lo, True), (pz_hi, True),
        (rh_lo, True), (rh_hi, True), (zg, True), (hl_lo, True),
        (hl_hi, True), *[(w, False) for w in wh]])


ADJ_BI = 1024
ADJ_BJ = 2048


def _tc_adj(z):
    def body(zi_r, zj_r, o_r):
        o_r[...] = jax.nn.sigmoid(
            lax.dot_general(zi_r[...], zj_r[...], (((1,), (1,)), ((), ()))))
    return pl.pallas_call(
        body, grid=(pl.cdiv(N, ADJ_BI), pl.cdiv(N, ADJ_BJ)),
        in_specs=[pl.BlockSpec((ADJ_BI, ZD), lambda i, j: (i, 0)),
                  pl.BlockSpec((ADJ_BJ, ZD), lambda i, j: (j, 0))],
        out_specs=pl.BlockSpec((ADJ_BI, ADJ_BJ), lambda i, j: (i, j)),
        out_shape=_f32((N, N)),
    )(z, z)


def kernel(x, h, edge_index, eps_noise, params):
    p = params
    hl_lo = h[0, :, :HH]
    hl_hi = h[0, :, HH:]
    src_e = edge_index[0]
    dst_e = edge_index[1]

    def b2(v):
        return v.reshape(1, -1)

    px_lo, px_hi = _tc_phiX(x, p["phi_x_W"], b2(p["phi_x_b"]))

    deg = _sc_deg(dst_e)
    Ap_lo, Ap_hi = _sc_seg(px_lo, px_hi, src_e, dst_e)
    Ah_lo, Ah_hi = _sc_seg(hl_lo, hl_hi, src_e, dst_e)

    ex_lo, ex_hi = _tc_encx(Ap_lo, Ap_hi, Ah_lo, Ah_hi, deg,
                            px_lo, px_hi, hl_lo, hl_hi,
                            p["enc_Wl"], b2(p["enc_bl"]), p["enc_Wr"])

    Ae_lo, Ae_hi = _sc_seg(ex_lo, ex_hi, src_e, dst_e)

    enc_mean, enc_std, prior_mean, prior_std, z, pz_lo, pz_hi = _tc_latent(
        Ae_lo, Ae_hi, deg, ex_lo, ex_hi, hl_lo, hl_hi, eps_noise,
        p["em_Wl"], b2(p["em_bl"]), p["em_Wr"],
        p["es_Wl"], b2(p["es_bl"]), p["es_Wr"],
        p["prior_W"], b2(p["prior_b"]),
        p["pm_W"], b2(p["pm_b"]), p["ps_W"], b2(p["ps_b"]),
        p["phi_z_W"], b2(p["phi_z_b"]))

    Az_lo, Az_hi = _sc_seg(pz_lo, pz_hi, src_e, dst_e)

    wz = [p["xz_Wl"], b2(p["xz_bl"]), p["xz_Wr"],
          p["hz_Wl"], b2(p["hz_bl"]), p["hz_Wr"]]
    wr = [p["xr_Wl"], b2(p["xr_bl"]), p["xr_Wr"],
          p["hr_Wl"], b2(p["hr_bl"]), p["hr_Wr"]]
    z_g, rh_lo, rh_hi = _tc_gates(Ap_lo, Ap_hi, Az_lo, Az_hi, Ah_lo, Ah_hi,
                                  deg, px_lo, px_hi, pz_lo, pz_hi,
                                  hl_lo, hl_hi, wz, wr)

    Ar_lo, Ar_hi = _sc_seg(rh_lo, rh_hi, src_e, dst_e)

    wh = [p["xh_Wl"], b2(p["xh_bl"]), p["xh_Wr"],
          p["hh_Wl"], b2(p["hh_bl"]), p["hh_Wr"]]
    out = _tc_hout(Ap_lo, Ap_hi, Az_lo, Az_hi, Ar_lo, Ar_hi, deg,
                   px_lo, px_hi, pz_lo, pz_hi, rh_lo, rh_hi, z_g,
                   hl_lo, hl_hi, wh)

    adj = _tc_adj(z)

    return (adj, prior_mean, prior_std, enc_mean, enc_std, z, out[None])
